# baseline (speedup 1.0000x reference)
import jax
import jax.numpy as jnp
from jax.experimental import pallas as pl
from jax.experimental.pallas import tpu as pltpu

_PATCH = 7
_STRIDE = 2
_ALPHA = 0.05
_NSIDE = 45
_N = _NSIDE * _NSIDE
_NPAD = 2048
_D = 147
_DPAD = 152
_BIG = 1.0e30


def _extract_fm(v):
    bsz = v.shape[0]
    hi = (_NSIDE - 1) * _STRIDE + 1
    cols = []
    for dy in range(_PATCH):
        for dx in range(_PATCH):
            s = v[:, :, dy:dy + hi:_STRIDE, dx:dx + hi:_STRIDE]
            cols.append(s.reshape(bsz, 3, _N))
    p = jnp.stack(cols, axis=1)
    return p.reshape(bsz, _D, _N)


def _loss_kernel(inp_ref, tgt_ref, out_ref):
    b = pl.program_id(0)
    nb = pl.num_programs(0)

    inp = inp_ref[0]
    tgt = tgt_ref[0]
    colid = jax.lax.broadcasted_iota(jnp.int32, (1, _NPAD), 1)

    xn = 0.5 * jnp.sum(inp * inp, axis=0, keepdims=True)
    yn = 0.5 * jnp.sum(tgt * tgt, axis=0, keepdims=True)
    inp_ref[0, _D:_D + 1, :] = jnp.where(colid < _N, xn, _BIG * 0.5)
    inp_ref[0, _D + 1:_D + 2, :] = jnp.ones((1, _NPAD), jnp.float32)
    tgt_ref[0, _D:_D + 1, :] = jnp.full((1, _NPAD), -1.0, jnp.float32)
    tgt_ref[0, _D + 1:_D + 2, :] = -yn

    dot = jax.lax.dot_general(
        tgt_ref[0], inp_ref[0], (((0,), (0,)), ((), ())),
        preferred_element_type=jnp.float32)

    rowmax = jnp.max(dot, axis=1, keepdims=True)
    denom = rowmax * (-2.0 / _D) + _ALPHA
    s = (-2.0 / _D) / denom
    rowid = jax.lax.broadcasted_iota(jnp.int32, (_NPAD, 1), 0)
    m = jnp.where(rowid < _N, 0.0, _BIG)
    cmin = jnp.min(dot * s + m, axis=0, keepdims=True)

    loss_b = jnp.sum(jnp.where(colid < _N, cmin, 0.0),
                     axis=1, keepdims=True) * (1.0 / _N)

    @pl.when(b == 0)
    def _init_out():
        out_ref[...] = jnp.zeros_like(out_ref)

    out_ref[...] += loss_b / nb


def kernel(x, y):
    xp = _extract_fm(x)
    yp = _extract_fm(y)
    bsz = xp.shape[0]
    xp = jnp.pad(xp, ((0, 0), (0, _DPAD - _D), (0, _NPAD - _N)))
    yp = jnp.pad(yp, ((0, 0), (0, _DPAD - _D), (0, _NPAD - _N)))

    out = pl.pallas_call(
        _loss_kernel,
        grid=(bsz,),
        in_specs=[
            pl.BlockSpec((1, _DPAD, _NPAD), lambda b: (b, 0, 0)),
            pl.BlockSpec((1, _DPAD, _NPAD), lambda b: (b, 0, 0)),
        ],
        out_specs=pl.BlockSpec((1, 1), lambda b: (0, 0)),
        out_shape=jax.ShapeDtypeStruct((1, 1), jnp.float32),
    )(xp, yp)
    return out[0, 0]


# speedup vs baseline: 3.2337x; 3.2337x over previous
import jax
import jax.numpy as jnp
from jax.experimental import pallas as pl

_PATCH = 7
_STRIDE = 2
_ALPHA = 0.05
_N = 2025
_NPAD = 2048
_D = 147
_DPAD = 152
_BIG = 1.0e30


def _loss_kernel(inp_ref, tgt_ref, out_ref):
    b = pl.program_id(0)
    nb = pl.num_programs(0)

    inp = inp_ref[0]
    tgt = tgt_ref[0]
    colid = jax.lax.broadcasted_iota(jnp.int32, (1, _NPAD), 1)

    xn = 0.5 * jnp.sum(inp * inp, axis=0, keepdims=True)
    yn = 0.5 * jnp.sum(tgt * tgt, axis=0, keepdims=True)
    inp_ref[0, _D:_D + 1, :] = jnp.where(colid < _N, xn, _BIG * 0.5)
    inp_ref[0, _D + 1:_D + 2, :] = jnp.ones((1, _NPAD), jnp.float32)
    tgt_ref[0, _D:_D + 1, :] = jnp.full((1, _NPAD), -1.0, jnp.float32)
    tgt_ref[0, _D + 1:_D + 2, :] = -yn

    dot = jax.lax.dot_general(
        tgt_ref[0], inp_ref[0], (((0,), (0,)), ((), ())),
        preferred_element_type=jnp.float32)

    rowmax = jnp.max(dot, axis=1, keepdims=True)
    denom = rowmax * (-2.0 / _D) + _ALPHA
    s = (-2.0 / _D) / denom
    rowid = jax.lax.broadcasted_iota(jnp.int32, (_NPAD, 1), 0)
    m = jnp.where(rowid < _N, 0.0, _BIG)
    cmin = jnp.min(dot * s + m, axis=0, keepdims=True)

    loss_b = jnp.sum(jnp.where(colid < _N, cmin, 0.0),
                     axis=1, keepdims=True) * (1.0 / _N)

    @pl.when(b == 0)
    def _init_out():
        out_ref[...] = jnp.zeros_like(out_ref)

    out_ref[...] += loss_b / nb


def kernel(x, y):
    def prep(v):
        p = jax.lax.conv_general_dilated_patches(
            v, filter_shape=(_PATCH, _PATCH), window_strides=(_STRIDE, _STRIDE),
            padding='VALID')
        bb, d, hh, ww = p.shape
        p = p.reshape(bb, d, hh * ww)
        return jnp.pad(p, ((0, 0), (0, _DPAD - _D), (0, _NPAD - _N)))

    xp = prep(x)
    yp = prep(y)
    bsz = xp.shape[0]

    out = pl.pallas_call(
        _loss_kernel,
        grid=(bsz,),
        in_specs=[
            pl.BlockSpec((1, _DPAD, _NPAD), lambda b: (b, 0, 0)),
            pl.BlockSpec((1, _DPAD, _NPAD), lambda b: (b, 0, 0)),
        ],
        out_specs=pl.BlockSpec((1, 1), lambda b: (0, 0)),
        out_shape=jax.ShapeDtypeStruct((1, 1), jnp.float32),
    )(xp, yp)
    return out[0, 0]


# speedup vs baseline: 3.4042x; 1.0527x over previous
import jax
import jax.numpy as jnp
import numpy as np
from jax.experimental import pallas as pl
from jax.experimental.pallas import tpu as pltpu

_PATCH = 7
_STRIDE = 2
_ALPHA = 0.05
_NSIDE = 45
_N = _NSIDE * _NSIDE
_GSIDE = 48
_NPAD = _GSIDE * _GSIDE
_IMG = 96
_IMGPAD = (_GSIDE - 1) * _STRIDE + _PATCH
_D = 147
_DPAD = 152
_BIG = 1.0e30

_valid = (np.arange(_NPAD) // _GSIDE < _NSIDE) & \
         (np.arange(_NPAD) % _GSIDE < _NSIDE)
_COLPAD = jnp.asarray(np.where(_valid, 0.0, _BIG)[None, :], jnp.float32)
_ROWPAD = jnp.asarray(np.where(_valid, 0.0, _BIG)[:, None], jnp.float32)
_COLSEL = jnp.asarray(np.where(_valid, 1.0, 0.0)[None, :], jnp.float32)


def _extract(v):
    vpad = jnp.pad(v, ((0, 0), (0, 0), (0, _IMGPAD - _IMG),
                       (0, _IMGPAD - _IMG)))
    p = jax.lax.conv_general_dilated_patches(
        vpad, filter_shape=(_PATCH, _PATCH), window_strides=(_STRIDE, _STRIDE),
        padding='VALID', dimension_numbers=('NCHW', 'OIHW', 'NHWC'))
    b = p.shape[0]
    p = p.reshape(b, _NPAD, _D)
    return jnp.pad(p, ((0, 0), (0, 0), (0, _DPAD - _D)))


def _loss_kernel(inp_ref, tgt_ref, rowpad_ref, colsel_ref, out_ref):
    b = pl.program_id(0)
    nb = pl.num_programs(0)

    inp = inp_ref[0]
    tgt = tgt_ref[0]

    xn = 0.5 * jnp.sum(inp * inp, axis=1, keepdims=True)
    yn = 0.5 * jnp.sum(tgt * tgt, axis=1, keepdims=True)
    inp_ref[0, :, _D:_D + 1] = xn + rowpad_ref[...] * 0.5
    inp_ref[0, :, _D + 1:_D + 2] = jnp.ones((_NPAD, 1), jnp.float32)
    tgt_ref[0, :, _D:_D + 1] = jnp.full((_NPAD, 1), -1.0, jnp.float32)
    tgt_ref[0, :, _D + 1:_D + 2] = -yn

    dot = jax.lax.dot_general(
        tgt_ref[0], inp_ref[0], (((1,), (1,)), ((), ())),
        preferred_element_type=jnp.float32)

    rowmax = jnp.max(dot, axis=1, keepdims=True)
    denom = rowmax * (-2.0 / _D) + _ALPHA
    s = (-2.0 / _D) / denom
    m = rowpad_ref[...]
    cmin = jnp.min(dot * s + m, axis=0, keepdims=True)

    loss_b = jnp.sum(cmin * colsel_ref[...],
                     axis=1, keepdims=True) * (1.0 / _N)

    @pl.when(b == 0)
    def _init_out():
        out_ref[...] = jnp.zeros_like(out_ref)

    out_ref[...] += loss_b / nb


def kernel(x, y):
    xp = _extract(x)
    yp = _extract(y)
    bsz = xp.shape[0]

    out = pl.pallas_call(
        _loss_kernel,
        grid=(bsz,),
        in_specs=[
            pl.BlockSpec((1, _NPAD, _DPAD), lambda b: (b, 0, 0)),
            pl.BlockSpec((1, _NPAD, _DPAD), lambda b: (b, 0, 0)),
            pl.BlockSpec((_NPAD, 1), lambda b: (0, 0)),
            pl.BlockSpec((1, _NPAD), lambda b: (0, 0)),
        ],
        out_specs=pl.BlockSpec((1, 1), lambda b: (0, 0)),
        out_shape=jax.ShapeDtypeStruct((1, 1), jnp.float32),
    )(xp, yp, _ROWPAD, _COLSEL)
    return out[0, 0]


# speedup vs baseline: 3.8870x; 1.1418x over previous
import jax
import jax.numpy as jnp
from jax.experimental import pallas as pl
from jax.experimental.pallas import tpu as pltpu

_PATCH = 7
_STRIDE = 2
_ALPHA = 0.05
_N = 2025
_NPAD = 2048
_D = 147
_DPAD = 256
_BIG = 1.0e30


def _extract_patches(x):
    patches = jax.lax.conv_general_dilated_patches(
        x, filter_shape=(_PATCH, _PATCH), window_strides=(_STRIDE, _STRIDE),
        padding='VALID')
    b, d, hh, ww = patches.shape
    return patches.reshape(b, d, hh * ww).transpose(0, 2, 1)


def _loss_kernel(inp_ref, tgt_ref, out_ref):
    b = pl.program_id(0)
    nb = pl.num_programs(0)

    inp = inp_ref[0]
    tgt = tgt_ref[0]
    rowid = jax.lax.broadcasted_iota(jnp.int32, (_NPAD, 1), 0)

    xn = 0.5 * jnp.sum(inp * inp, axis=1, keepdims=True)
    yn = 0.5 * jnp.sum(tgt * tgt, axis=1, keepdims=True)
    inp_ref[0, :, _D:_D + 1] = jnp.where(rowid < _N, xn, _BIG * 0.5)
    inp_ref[0, :, _D + 1:_D + 2] = jnp.ones((_NPAD, 1), jnp.float32)
    tgt_ref[0, :, _D:_D + 1] = jnp.full((_NPAD, 1), -1.0, jnp.float32)
    tgt_ref[0, :, _D + 1:_D + 2] = -yn

    dot = jax.lax.dot_general(
        tgt_ref[0], inp_ref[0], (((1,), (1,)), ((), ())),
        preferred_element_type=jnp.float32)

    rowmax = jnp.max(dot, axis=1, keepdims=True)
    denom = rowmax * (-2.0 / _D) + _ALPHA
    s = (-2.0 / _D) / denom
    m = jnp.where(rowid < _N, 0.0, _BIG)
    cmin = jnp.min(dot * s + m, axis=0, keepdims=True)

    colmask = jax.lax.broadcasted_iota(jnp.int32, (1, _NPAD), 1) < _N
    loss_b = jnp.sum(jnp.where(colmask, cmin, 0.0),
                     axis=1, keepdims=True) * (1.0 / _N)

    @pl.when(b == 0)
    def _init_out():
        out_ref[...] = jnp.zeros_like(out_ref)

    out_ref[...] += loss_b / nb


def kernel(x, y):
    xp = _extract_patches(x)
    yp = _extract_patches(y)
    bsz = xp.shape[0]
    xp = jnp.pad(xp, ((0, 0), (0, _NPAD - _N), (0, _DPAD - _D)))
    yp = jnp.pad(yp, ((0, 0), (0, _NPAD - _N), (0, _DPAD - _D)))

    out = pl.pallas_call(
        _loss_kernel,
        grid=(bsz,),
        in_specs=[
            pl.BlockSpec((1, _NPAD, _DPAD), lambda b: (b, 0, 0)),
            pl.BlockSpec((1, _NPAD, _DPAD), lambda b: (b, 0, 0)),
        ],
        out_specs=pl.BlockSpec((1, 1), lambda b: (0, 0)),
        out_shape=jax.ShapeDtypeStruct((1, 1), jnp.float32),
    )(xp, yp)
    return out[0, 0]
